# baseline (speedup 1.0000x reference)
import functools

import jax
import jax.numpy as jnp
from jax.experimental import pallas as pl
from jax.experimental.pallas import tpu as pltpu

_E = 8
_C = 10
_DIN = 768
_DF = 768
_DOUT = 64
_TB = 1024


def _fused_body(state_ref, c_ref, wtask_ref, w1_ref, b1_ref, w2_ref, b2_ref,
                hw_ref, hb_ref, out_ref):
    c_col = c_ref[0]
    wt = wtask_ref[...]
    oh = c_col == jax.lax.broadcasted_iota(jnp.int32, (_TB, _C), 1)
    x = state_ref[...].astype(jnp.bfloat16)

    acc = jnp.zeros((_TB, _DF), jnp.float32)
    for e in range(_E):
        h = jnp.dot(x, w1_ref[e], preferred_element_type=jnp.float32)
        h = jnp.maximum(h + b1_ref[e], 0.0).astype(jnp.bfloat16)
        f = jnp.dot(h, w2_ref[e], preferred_element_type=jnp.float32)
        f = jnp.maximum(f + b2_ref[e], 0.0)
        gate = jnp.sum(jnp.where(oh, wt[e:e + 1, :], 0.0),
                       axis=1, keepdims=True)
        acc = acc + gate * f

    mixed = jnp.maximum(acc, 0.0).astype(jnp.bfloat16)
    all_heads = jnp.dot(mixed, hw_ref[...],
                        preferred_element_type=jnp.float32)
    all_heads = all_heads + hb_ref[...]
    out = jnp.zeros((_TB, _DOUT), jnp.float32)
    for ci in range(_C):
        sl = all_heads[:, ci * _DOUT:(ci + 1) * _DOUT]
        out = out + jnp.where(c_col == ci, sl, 0.0)
    out_ref[...] = out


@functools.partial(jax.jit, static_argnames=())
def kernel(state, c, W_task, W1, b1, W2, b2, head_W, head_b):
    B = state.shape[0]
    nb = B // _TB
    c3 = c.astype(jnp.int32).reshape(nb, _TB, 1)
    hw_cat = jnp.transpose(head_W, (1, 0, 2)).reshape(_DF, _C * _DOUT)
    hb_cat = head_b.reshape(1, _C * _DOUT)

    out = pl.pallas_call(
        _fused_body,
        grid=(nb,),
        in_specs=[
            pl.BlockSpec((_TB, _DIN), lambda ib: (ib, 0)),
            pl.BlockSpec((1, _TB, 1), lambda ib: (ib, 0, 0)),
            pl.BlockSpec((_E, _C), lambda ib: (0, 0)),
            pl.BlockSpec((_E, _DIN, _DF), lambda ib: (0, 0, 0)),
            pl.BlockSpec((_E, 1, _DF), lambda ib: (0, 0, 0)),
            pl.BlockSpec((_E, _DF, _DF), lambda ib: (0, 0, 0)),
            pl.BlockSpec((_E, 1, _DF), lambda ib: (0, 0, 0)),
            pl.BlockSpec((_DF, _C * _DOUT), lambda ib: (0, 0)),
            pl.BlockSpec((1, _C * _DOUT), lambda ib: (0, 0)),
        ],
        out_specs=pl.BlockSpec((_TB, _DOUT), lambda ib: (ib, 0)),
        out_shape=jax.ShapeDtypeStruct((B, _DOUT), jnp.float32),
        compiler_params=pltpu.CompilerParams(
            dimension_semantics=("arbitrary",),
        ),
    )(
        state,
        c3,
        W_task,
        W1.astype(jnp.bfloat16),
        b1.reshape(_E, 1, _DF),
        W2.astype(jnp.bfloat16),
        b2.reshape(_E, 1, _DF),
        hw_cat.astype(jnp.bfloat16),
        hb_cat,
    )
    return out


# speedup vs baseline: 1.1177x; 1.1177x over previous
import functools

import jax
import jax.numpy as jnp
from jax.experimental import pallas as pl
from jax.experimental.pallas import tpu as pltpu

_E = 8
_C = 10
_DIN = 768
_DF = 768
_DOUT = 64
_TB = 1024


def _fused_body(state_ref, c_ref, wtask_ref, w1_ref, b1_ref, w2_ref, b2_ref,
                hw_ref, hb_ref, out_ref):
    c_col = c_ref[0]
    wt = wtask_ref[...]
    oh = c_col == jax.lax.broadcasted_iota(jnp.int32, (_TB, _C), 1)
    x = state_ref[...]

    acc = jnp.zeros((_TB, _DF), jnp.float32)
    for e in range(_E):
        h = jnp.dot(x, w1_ref[e], preferred_element_type=jnp.float32)
        h = jnp.maximum(h + b1_ref[e], 0.0)
        f = jnp.dot(h, w2_ref[e], preferred_element_type=jnp.float32)
        f = jnp.maximum(f + b2_ref[e], 0.0)
        gate = jnp.sum(jnp.where(oh, wt[e:e + 1, :], 0.0),
                       axis=1, keepdims=True)
        acc = acc + gate * f

    mixed = jnp.maximum(acc, 0.0)
    all_heads = jnp.dot(mixed, hw_ref[...],
                        preferred_element_type=jnp.float32)
    all_heads = all_heads + hb_ref[...]
    out = jnp.zeros((_TB, _DOUT), jnp.float32)
    for ci in range(_C):
        sl = all_heads[:, ci * _DOUT:(ci + 1) * _DOUT]
        out = out + jnp.where(c_col == ci, sl, 0.0)
    out_ref[...] = out


@functools.partial(jax.jit, static_argnames=())
def kernel(state, c, W_task, W1, b1, W2, b2, head_W, head_b):
    B = state.shape[0]
    nb = B // _TB
    c3 = c.astype(jnp.int32).reshape(nb, _TB, 1)
    hw_cat = jnp.transpose(head_W, (1, 0, 2)).reshape(_DF, _C * _DOUT)
    hb_cat = head_b.reshape(1, _C * _DOUT)

    out = pl.pallas_call(
        _fused_body,
        grid=(nb,),
        in_specs=[
            pl.BlockSpec((_TB, _DIN), lambda ib: (ib, 0)),
            pl.BlockSpec((1, _TB, 1), lambda ib: (ib, 0, 0)),
            pl.BlockSpec((_E, _C), lambda ib: (0, 0)),
            pl.BlockSpec((_E, _DIN, _DF), lambda ib: (0, 0, 0)),
            pl.BlockSpec((_E, 1, _DF), lambda ib: (0, 0, 0)),
            pl.BlockSpec((_E, _DF, _DF), lambda ib: (0, 0, 0)),
            pl.BlockSpec((_E, 1, _DF), lambda ib: (0, 0, 0)),
            pl.BlockSpec((_DF, _C * _DOUT), lambda ib: (0, 0)),
            pl.BlockSpec((1, _C * _DOUT), lambda ib: (0, 0)),
        ],
        out_specs=pl.BlockSpec((_TB, _DOUT), lambda ib: (ib, 0)),
        out_shape=jax.ShapeDtypeStruct((B, _DOUT), jnp.float32),
        compiler_params=pltpu.CompilerParams(
            dimension_semantics=("arbitrary",),
        ),
    )(
        state,
        c3,
        W_task,
        W1,
        b1.reshape(_E, 1, _DF),
        W2,
        b2.reshape(_E, 1, _DF),
        hw_cat,
        hb_cat,
    )
    return out


# speedup vs baseline: 1.1189x; 1.0011x over previous
import functools

import jax
import jax.numpy as jnp
from jax.experimental import pallas as pl
from jax.experimental.pallas import tpu as pltpu

_E = 8
_C = 10
_DIN = 768
_DF = 768
_DOUT = 64
_TB = 1024


def _fused_body(state_ref, c_ref, wtask_ref, w1_ref, b1_ref, w2_ref, b2_ref,
                hw_ref, hb_ref, out_ref):
    c_col = c_ref[0]
    wt = wtask_ref[...]
    oh = c_col == jax.lax.broadcasted_iota(jnp.int32, (_TB, _C), 1)
    x = state_ref[...]

    acc = jnp.zeros((_TB, _DF), jnp.float32)
    for e in range(_E):
        h = jnp.dot(x, w1_ref[e], preferred_element_type=jnp.float32)
        h = jnp.maximum(h + b1_ref[e], 0.0)
        f = jnp.dot(h, w2_ref[e], preferred_element_type=jnp.float32)
        f = jnp.maximum(f + b2_ref[e], 0.0)
        gate = jnp.sum(jnp.where(oh, wt[e:e + 1, :], 0.0),
                       axis=1, keepdims=True)
        acc = acc + gate * f

    mixed = jnp.maximum(acc, 0.0)
    all_heads = jnp.dot(mixed, hw_ref[...],
                        preferred_element_type=jnp.float32)
    all_heads = all_heads + hb_ref[...]
    out = jnp.zeros((_TB, _DOUT), jnp.float32)
    for ci in range(_C):
        sl = all_heads[:, ci * _DOUT:(ci + 1) * _DOUT]
        out = out + jnp.where(c_col == ci, sl, 0.0)
    out_ref[...] = out


@functools.partial(jax.jit, static_argnames=())
def kernel(state, c, W_task, W1, b1, W2, b2, head_W, head_b):
    B = state.shape[0]
    nb = B // _TB
    c3 = c.astype(jnp.int32).reshape(nb, _TB, 1)
    hw_cat = jnp.transpose(head_W, (1, 0, 2)).reshape(_DF, _C * _DOUT)
    hb_cat = head_b.reshape(1, _C * _DOUT)

    out = pl.pallas_call(
        _fused_body,
        grid=(nb,),
        in_specs=[
            pl.BlockSpec((_TB, _DIN), lambda ib: (ib, 0)),
            pl.BlockSpec((1, _TB, 1), lambda ib: (ib, 0, 0)),
            pl.BlockSpec((_E, _C), lambda ib: (0, 0)),
            pl.BlockSpec((_E, _DIN, _DF), lambda ib: (0, 0, 0)),
            pl.BlockSpec((_E, 1, _DF), lambda ib: (0, 0, 0)),
            pl.BlockSpec((_E, _DF, _DF), lambda ib: (0, 0, 0)),
            pl.BlockSpec((_E, 1, _DF), lambda ib: (0, 0, 0)),
            pl.BlockSpec((_DF, _C * _DOUT), lambda ib: (0, 0)),
            pl.BlockSpec((1, _C * _DOUT), lambda ib: (0, 0)),
        ],
        out_specs=pl.BlockSpec((_TB, _DOUT), lambda ib: (ib, 0)),
        out_shape=jax.ShapeDtypeStruct((B, _DOUT), jnp.float32),
        compiler_params=pltpu.CompilerParams(
            dimension_semantics=("parallel",),
        ),
    )(
        state,
        c3,
        W_task,
        W1,
        b1.reshape(_E, 1, _DF),
        W2,
        b2.reshape(_E, 1, _DF),
        hw_cat,
        hb_cat,
    )
    return out
